# Initial kernel scaffold; baseline (speedup 1.0000x reference)
import jax
import jax.numpy as jnp
from jax.experimental import pallas as pl


def kernel(x, table):
    raise NotImplementedError("write your pallas kernel here")


# speedup vs baseline: 2.7949x; 2.7949x over previous
import functools

import jax
import jax.numpy as jnp
from jax import lax
from jax.experimental import pallas as pl
from jax.experimental.pallas import tpu as pltpu
from jax.experimental.pallas import tpu_sc as plsc

_B = 16384
_L = 200
_H = 64
_BT = _B * _L
_NP = _BT // 2
_NW = 32
_PPW = _NP // _NW
_PCHUNK = 512
_NIDX = 128
_JOPS = _PCHUNK // _NIDX
_NCH = _PPW // _PCHUNK


def _build():
    mesh = plsc.VectorSubcoreMesh(core_axis_name="c", subcore_axis_name="s")

    @functools.partial(
        pl.kernel,
        mesh=mesh,
        out_type=jax.ShapeDtypeStruct((_NP, 2 * _H), jnp.float32),
        scratch_types=[
            pltpu.VMEM((_PCHUNK,), jnp.int32),
            pltpu.VMEM((_PCHUNK,), jnp.int32),
            pltpu.VMEM((_JOPS, _NIDX), jnp.int32),
            pltpu.VMEM((_PCHUNK, 2 * _H), jnp.float32),
            pltpu.SemaphoreType.DMA,
        ],
    )
    def k(tab2_hbm, ev_hbm, od_hbm, out_hbm, ev_v, od_v, pidx_v, rows_v, sem):
        wid = lax.axis_index("s") * 2 + lax.axis_index("c")
        pair_base = wid * _PPW

        def body(c, carry):
            pair_off = pair_base + c * _PCHUNK
            pltpu.sync_copy(ev_hbm.at[pl.ds(pair_off, _PCHUNK)], ev_v)
            pltpu.sync_copy(od_hbm.at[pl.ds(pair_off, _PCHUNK)], od_v)
            for j in range(_JOPS):
                for g in range(_NIDX // 16):
                    s = pl.ds(j * _NIDX + g * 16, 16)
                    pidx_v[j, pl.ds(g * 16, 16)] = ev_v[s] * 10 + od_v[s]
            copies = []
            for j in range(_JOPS):
                copies.append(
                    pltpu.async_copy(
                        tab2_hbm.at[pidx_v.at[j]],
                        rows_v.at[pl.ds(j * _NIDX, _NIDX)],
                        sem,
                    )
                )
            for cp in copies:
                cp.wait()
            pltpu.sync_copy(rows_v, out_hbm.at[pl.ds(pair_off, _PCHUNK)])
            return carry

        lax.fori_loop(0, _NCH, body, 0)

    return k


_kernel_call = _build()


def kernel(x, table):
    idx = x.reshape(_NP, 2).astype(jnp.int32)
    tab2 = jnp.concatenate(
        [jnp.repeat(table, 10, axis=0), jnp.tile(table, (10, 1))], axis=1
    )
    out = _kernel_call(tab2, idx[:, 0], idx[:, 1])
    return out.reshape(_B, _L, _H)


# speedup vs baseline: 2.8027x; 1.0028x over previous
import functools

import jax
import jax.numpy as jnp
from jax import lax
from jax.experimental import pallas as pl
from jax.experimental.pallas import tpu as pltpu
from jax.experimental.pallas import tpu_sc as plsc

_B = 16384
_L = 200
_H = 64
_BT = _B * _L
_NP = _BT // 2
_NW = 32
_PPW = _NP // _NW
_PCHUNK = 256
_NIDX = 128
_JOPS = _PCHUNK // _NIDX
_NCH = _PPW // _PCHUNK


def _build():
    mesh = plsc.VectorSubcoreMesh(core_axis_name="c", subcore_axis_name="s")

    @functools.partial(
        pl.kernel,
        mesh=mesh,
        out_type=jax.ShapeDtypeStruct((_NP, 2 * _H), jnp.float32),
        scratch_types=[
            pltpu.VMEM((_PCHUNK,), jnp.int32),
            pltpu.VMEM((_PCHUNK,), jnp.int32),
            pltpu.VMEM((_PCHUNK,), jnp.int32),
            pltpu.VMEM((_PCHUNK,), jnp.int32),
            pltpu.VMEM((_JOPS, _NIDX), jnp.int32),
            pltpu.VMEM((_JOPS, _NIDX), jnp.int32),
            pltpu.VMEM((_PCHUNK, 2 * _H), jnp.float32),
            pltpu.VMEM((_PCHUNK, 2 * _H), jnp.float32),
            pltpu.SemaphoreType.DMA,
            pltpu.SemaphoreType.DMA,
            pltpu.SemaphoreType.DMA,
            pltpu.SemaphoreType.DMA,
        ],
    )
    def k(tab2_hbm, ev_hbm, od_hbm, out_hbm,
          ev0, ev1, od0, od1, pidx0, pidx1, rows0, rows1,
          sg0, sg1, so0, so1):
        evb = (ev0, ev1)
        odb = (od0, od1)
        pidxb = (pidx0, pidx1)
        rows = (rows0, rows1)
        sg = (sg0, sg1)
        so = (so0, so1)
        wid = lax.axis_index("s") * 2 + lax.axis_index("c")
        pair_base = wid * _PPW

        def load_and_pidx(c, b):
            off = pair_base + c * _PCHUNK
            pltpu.sync_copy(ev_hbm.at[pl.ds(off, _PCHUNK)], evb[b])
            pltpu.sync_copy(od_hbm.at[pl.ds(off, _PCHUNK)], odb[b])
            for j in range(_JOPS):
                for g in range(_NIDX // 16):
                    s = pl.ds(j * _NIDX + g * 16, 16)
                    pidxb[b][j, pl.ds(g * 16, 16)] = (
                        evb[b][s] * 10 + odb[b][s]
                    )

        def start_gather(b):
            for j in range(_JOPS):
                pltpu.async_copy(
                    tab2_hbm.at[pidxb[b].at[j]],
                    rows[b].at[pl.ds(j * _NIDX, _NIDX)],
                    sg[b],
                )

        def wait_rows(b, sem):
            pltpu.make_async_copy(
                out_hbm.at[pl.ds(0, _PCHUNK)], rows[b], sem
            ).wait()

        def start_write(c, b):
            off = pair_base + c * _PCHUNK
            pltpu.async_copy(rows[b], out_hbm.at[pl.ds(off, _PCHUNK)], so[b])

        for b in (0, 1):
            load_and_pidx(b, b)
            start_gather(b)

        def body(c2, carry):
            for b in (0, 1):
                c = 2 * c2 + b
                wait_rows(b, sg[b])
                start_write(c, b)
                load_and_pidx(c + 2, b)
                wait_rows(b, so[b])
                start_gather(b)
            return carry

        lax.fori_loop(0, _NCH // 2 - 1, body, 0)

        for b in (0, 1):
            wait_rows(b, sg[b])
            start_write(_NCH - 2 + b, b)
        for b in (0, 1):
            wait_rows(b, so[b])

    return k


_kernel_call = _build()


def kernel(x, table):
    idx = x.reshape(_NP, 2).astype(jnp.int32)
    tab2 = jnp.concatenate(
        [jnp.repeat(table, 10, axis=0), jnp.tile(table, (10, 1))], axis=1
    )
    out = _kernel_call(tab2, idx[:, 0], idx[:, 1])
    return out.reshape(_B, _L, _H)


# speedup vs baseline: 5.8265x; 2.0789x over previous
import functools

import jax
import jax.numpy as jnp
from jax import lax
from jax.experimental import pallas as pl
from jax.experimental.pallas import tpu as pltpu
from jax.experimental.pallas import tpu_sc as plsc

_B = 16384
_L = 200
_H = 64
_BT = _B * _L
_NP = _BT // 2
_NW = 32
_PPW = _NP // _NW
_PCHUNK = 256
_NIDX = 128
_JOPS = _PCHUNK // _NIDX
_NCH = _PPW // _PCHUNK

_DNUMS = lax.GatherDimensionNumbers(
    offset_dims=(), collapsed_slice_dims=(0,), start_index_map=(0,)
)


def _dg(v, perm):
    return lax.gather(
        v, perm.reshape(16, 1), _DNUMS, (1,),
        mode=lax.GatherScatterMode.PROMISE_IN_BOUNDS,
    )


def _build():
    mesh = plsc.VectorSubcoreMesh(core_axis_name="c", subcore_axis_name="s")

    @functools.partial(
        pl.kernel,
        mesh=mesh,
        out_type=jax.ShapeDtypeStruct((_NP, 2 * _H), jnp.float32),
        scratch_types=[
            pltpu.VMEM_SHARED((100, 2 * _H), jnp.float32),
            pltpu.VMEM((2 * _PCHUNK,), jnp.int32),
            pltpu.VMEM((2 * _PCHUNK,), jnp.int32),
            pltpu.VMEM((_JOPS, _NIDX), jnp.int32),
            pltpu.VMEM((_JOPS, _NIDX), jnp.int32),
            pltpu.VMEM((_PCHUNK, 2 * _H), jnp.float32),
            pltpu.VMEM((_PCHUNK, 2 * _H), jnp.float32),
            pltpu.SemaphoreType.DMA,
            pltpu.SemaphoreType.DMA,
            pltpu.SemaphoreType.DMA,
            pltpu.SemaphoreType.DMA,
        ],
    )
    def k(tab2_hbm, x_hbm, out_hbm,
          tab_sh, raw0, raw1, pidx0, pidx1, rows0, rows1,
          sg0, sg1, so0, so1):
        rawb = (raw0, raw1)
        pidxb = (pidx0, pidx1)
        rows = (rows0, rows1)
        sg = (sg0, sg1)
        so = (so0, so1)
        cid = lax.axis_index("c")
        sid = lax.axis_index("s")
        wid = sid * 2 + cid
        pair_base = wid * _PPW

        @pl.when(sid == 0)
        def _stage_table():
            pltpu.sync_copy(tab2_hbm, tab_sh)

        plsc.subcore_barrier()

        iota = lax.iota(jnp.int32, 16)
        perm_e = (iota % 8) * 2
        perm_o = perm_e + 1
        lo = iota < 8

        def load_and_pidx(c, b):
            off = pair_base + c * _PCHUNK
            pltpu.sync_copy(
                x_hbm.at[pl.ds(off * 2, 2 * _PCHUNK)], rawb[b]
            )
            for j in range(_JOPS):
                for g in range(_NIDX // 16):
                    q = j * _NIDX + g * 16
                    a = rawb[b][pl.ds(2 * q, 16)]
                    bb = rawb[b][pl.ds(2 * q + 16, 16)]
                    ev = jnp.where(lo, _dg(a, perm_e), _dg(bb, perm_e))
                    od = jnp.where(lo, _dg(a, perm_o), _dg(bb, perm_o))
                    pidxb[b][j, pl.ds(g * 16, 16)] = ev * 10 + od

        def start_gather(b):
            for j in range(_JOPS):
                pltpu.async_copy(
                    tab_sh.at[pidxb[b].at[j]],
                    rows[b].at[pl.ds(j * _NIDX, _NIDX)],
                    sg[b],
                )

        def wait_rows(b, sem):
            pltpu.make_async_copy(
                out_hbm.at[pl.ds(0, _PCHUNK)], rows[b], sem
            ).wait()

        def start_write(c, b):
            off = pair_base + c * _PCHUNK
            pltpu.async_copy(rows[b], out_hbm.at[pl.ds(off, _PCHUNK)], so[b])

        for b in (0, 1):
            load_and_pidx(b, b)
            start_gather(b)

        def body(c2, carry):
            for b in (0, 1):
                c = 2 * c2 + b
                wait_rows(b, sg[b])
                start_write(c, b)
                load_and_pidx(c + 2, b)
                wait_rows(b, so[b])
                start_gather(b)
            return carry

        lax.fori_loop(0, _NCH // 2 - 1, body, 0)

        for b in (0, 1):
            wait_rows(b, sg[b])
            start_write(_NCH - 2 + b, b)
        for b in (0, 1):
            wait_rows(b, so[b])

    return k


_kernel_call = _build()


def kernel(x, table):
    tab2 = jnp.concatenate(
        [jnp.repeat(table, 10, axis=0), jnp.tile(table, (10, 1))], axis=1
    )
    out = _kernel_call(tab2, x.reshape(_BT).astype(jnp.int32))
    return out.reshape(_B, _L, _H)
